# baseline (speedup 1.0000x reference)
import functools

import jax
import jax.numpy as jnp
from jax.experimental import pallas as pl
from jax.experimental.pallas import tpu as pltpu

NUM_EMB = 1024
DIM = 256
TOKENS = 32 * 1024
BT = 1024


def _prep_kernel(w_ref, w2_ref, whi_ref, wlo_ref):
    w = w_ref[...]
    w2_ref[...] = jnp.sum(w * w, axis=1)[None, :]
    w_hi = w.astype(jnp.bfloat16)
    whi_ref[...] = w_hi
    wlo_ref[...] = (w - w_hi.astype(jnp.float32)).astype(jnp.bfloat16)


def _vq_kernel(x_ref, w2_ref, whi_ref, wlo_ref, enc_ref, q_ref,
               loss_ref, perp_ref, sumsq_ref, counts_ref):
    i = pl.program_id(0)
    nsteps = pl.num_programs(0)
    x = x_ref[...]

    xw = jax.lax.dot_general(
        x.astype(jnp.bfloat16), whi_ref[...],
        dimension_numbers=(((1,), (1,)), ((), ())),
        preferred_element_type=jnp.float32)
    x2 = jnp.sum(x * x, axis=1, keepdims=True)
    dist = x2 + w2_ref[...] - 2.0 * xw
    code_iota = jax.lax.broadcasted_iota(jnp.int32, (BT, NUM_EMB), 1)
    mn = jnp.min(dist, axis=1, keepdims=True)
    masked_iota = jnp.where(dist == mn, code_iota, NUM_EMB)
    idx = jnp.min(masked_iota, axis=1)

    onehot = (code_iota == idx[:, None]).astype(jnp.float32)
    enc_ref[...] = onehot

    ob = onehot.astype(jnp.bfloat16)
    dims = (((1,), (0,)), ((), ()))
    q = (jax.lax.dot_general(ob, whi_ref[...], dims,
                             preferred_element_type=jnp.float32)
         + jax.lax.dot_general(ob, wlo_ref[...], dims,
                               preferred_element_type=jnp.float32))
    q_ref[...] = x + (q - x)

    blk_sumsq = jnp.sum(mn)
    blk_counts = jnp.sum(onehot, axis=0)[None, :]

    @pl.when(i == 0)
    def _init():
        sumsq_ref[0, 0] = blk_sumsq
        counts_ref[...] = blk_counts

    @pl.when(i > 0)
    def _acc():
        sumsq_ref[0, 0] += blk_sumsq
        counts_ref[...] += blk_counts

    @pl.when(i == nsteps - 1)
    def _finalize():
        n_elem = jnp.float32(TOKENS * DIM)
        loss_ref[...] = jnp.full((1, 1), 1.25 * sumsq_ref[0, 0] / n_elem,
                                 dtype=jnp.float32)
        probs = counts_ref[...] / jnp.float32(TOKENS)
        ent = jnp.sum(probs * jnp.log(probs + 1e-10))
        perp_ref[...] = jnp.full((1, 1), jnp.exp(-ent), dtype=jnp.float32)


@functools.partial(jax.jit, static_argnames=())
def _vq(flat_x, weight):
    w2, w_hi, w_lo = pl.pallas_call(
        _prep_kernel,
        out_shape=[
            jax.ShapeDtypeStruct((1, NUM_EMB), jnp.float32),
            jax.ShapeDtypeStruct((NUM_EMB, DIM), jnp.bfloat16),
            jax.ShapeDtypeStruct((NUM_EMB, DIM), jnp.bfloat16),
        ],
    )(weight)

    grid = (TOKENS // BT,)
    enc, q, loss, perp = pl.pallas_call(
        _vq_kernel,
        grid=grid,
        in_specs=[
            pl.BlockSpec((BT, DIM), lambda i: (i, 0)),
            pl.BlockSpec((1, NUM_EMB), lambda i: (0, 0)),
            pl.BlockSpec((NUM_EMB, DIM), lambda i: (0, 0)),
            pl.BlockSpec((NUM_EMB, DIM), lambda i: (0, 0)),
        ],
        out_specs=[
            pl.BlockSpec((BT, NUM_EMB), lambda i: (i, 0)),
            pl.BlockSpec((BT, DIM), lambda i: (i, 0)),
            pl.BlockSpec((1, 1), lambda i: (0, 0)),
            pl.BlockSpec((1, 1), lambda i: (0, 0)),
        ],
        out_shape=[
            jax.ShapeDtypeStruct((TOKENS, NUM_EMB), jnp.float32),
            jax.ShapeDtypeStruct((TOKENS, DIM), jnp.float32),
            jax.ShapeDtypeStruct((1, 1), jnp.float32),
            jax.ShapeDtypeStruct((1, 1), jnp.float32),
        ],
        scratch_shapes=[
            pltpu.SMEM((1, 1), jnp.float32),
            pltpu.VMEM((1, NUM_EMB), jnp.float32),
        ],
    )(flat_x, w2, w_hi, w_lo)
    return enc, q, loss, perp


def kernel(inputs, weight):
    flat_x = inputs.reshape(-1, DIM)
    enc, q, loss, perp = _vq(flat_x, weight)
    quantized = q.reshape(inputs.shape)
    return (loss[0, 0], quantized, perp[0, 0], enc)


# speedup vs baseline: 1.1909x; 1.1909x over previous
import functools

import jax
import jax.numpy as jnp
from jax.experimental import pallas as pl
from jax.experimental.pallas import tpu as pltpu

NUM_EMB = 1024
DIM = 256
TOKENS = 32 * 1024
BT = 1024


def _prep_kernel(w_ref, w2_ref, whi_ref):
    w = w_ref[...]
    w2_ref[...] = jnp.sum(w * w, axis=1)[None, :]
    whi_ref[...] = w.astype(jnp.bfloat16)


def _vq_kernel(x_ref, w2_ref, whi_ref, enc_ref, q_ref,
               loss_ref, perp_ref, sumsq_ref, counts_ref):
    i = pl.program_id(0)
    nsteps = pl.num_programs(0)
    x = x_ref[...]

    xw = jax.lax.dot_general(
        x.astype(jnp.bfloat16), whi_ref[...],
        dimension_numbers=(((1,), (1,)), ((), ())),
        preferred_element_type=jnp.float32)
    x2 = jnp.sum(x * x, axis=1, keepdims=True)
    dist = x2 + w2_ref[...] - 2.0 * xw
    code_iota = jax.lax.broadcasted_iota(jnp.int32, (BT, NUM_EMB), 1)
    mn = jnp.min(dist, axis=1, keepdims=True)
    masked_iota = jnp.where(dist == mn, code_iota, NUM_EMB)
    idx = jnp.min(masked_iota, axis=1)

    onehot = (code_iota == idx[:, None]).astype(jnp.float32)
    enc_ref[...] = onehot

    ob = onehot.astype(jnp.bfloat16)
    dims = (((1,), (0,)), ((), ()))
    q = jax.lax.dot_general(ob, whi_ref[...], dims,
                            preferred_element_type=jnp.float32)
    q_ref[...] = x + (q - x)

    blk_sumsq = jnp.sum(mn)
    blk_counts = jnp.sum(onehot, axis=0)[None, :]

    @pl.when(i == 0)
    def _init():
        sumsq_ref[0, 0] = blk_sumsq
        counts_ref[...] = blk_counts

    @pl.when(i > 0)
    def _acc():
        sumsq_ref[0, 0] += blk_sumsq
        counts_ref[...] += blk_counts

    @pl.when(i == nsteps - 1)
    def _finalize():
        n_elem = jnp.float32(TOKENS * DIM)
        loss_ref[...] = jnp.full((1, 1), 1.25 * sumsq_ref[0, 0] / n_elem,
                                 dtype=jnp.float32)
        probs = counts_ref[...] / jnp.float32(TOKENS)
        ent = jnp.sum(probs * jnp.log(probs + 1e-10))
        perp_ref[...] = jnp.full((1, 1), jnp.exp(-ent), dtype=jnp.float32)


@functools.partial(jax.jit, static_argnames=())
def _vq(flat_x, weight):
    w2, w_hi = pl.pallas_call(
        _prep_kernel,
        out_shape=[
            jax.ShapeDtypeStruct((1, NUM_EMB), jnp.float32),
            jax.ShapeDtypeStruct((NUM_EMB, DIM), jnp.bfloat16),
        ],
    )(weight)

    grid = (TOKENS // BT,)
    enc, q, loss, perp = pl.pallas_call(
        _vq_kernel,
        grid=grid,
        in_specs=[
            pl.BlockSpec((BT, DIM), lambda i: (i, 0)),
            pl.BlockSpec((1, NUM_EMB), lambda i: (0, 0)),
            pl.BlockSpec((NUM_EMB, DIM), lambda i: (0, 0)),
        ],
        out_specs=[
            pl.BlockSpec((BT, NUM_EMB), lambda i: (i, 0)),
            pl.BlockSpec((BT, DIM), lambda i: (i, 0)),
            pl.BlockSpec((1, 1), lambda i: (0, 0)),
            pl.BlockSpec((1, 1), lambda i: (0, 0)),
        ],
        out_shape=[
            jax.ShapeDtypeStruct((TOKENS, NUM_EMB), jnp.float32),
            jax.ShapeDtypeStruct((TOKENS, DIM), jnp.float32),
            jax.ShapeDtypeStruct((1, 1), jnp.float32),
            jax.ShapeDtypeStruct((1, 1), jnp.float32),
        ],
        scratch_shapes=[
            pltpu.SMEM((1, 1), jnp.float32),
            pltpu.VMEM((1, NUM_EMB), jnp.float32),
        ],
    )(flat_x, w2, w_hi)
    return enc, q, loss, perp


def kernel(inputs, weight):
    flat_x = inputs.reshape(-1, DIM)
    enc, q, loss, perp = _vq(flat_x, weight)
    quantized = q.reshape(inputs.shape)
    return (loss[0, 0], quantized, perp[0, 0], enc)


# speedup vs baseline: 1.2734x; 1.0693x over previous
import functools

import jax
import jax.numpy as jnp
from jax.experimental import pallas as pl
from jax.experimental.pallas import tpu as pltpu

NUM_EMB = 1024
DIM = 256
TOKENS = 32 * 1024
BT = 2048


def _prep_kernel(w_ref, w2_ref, whi_ref):
    w = w_ref[...]
    w2_ref[...] = jnp.sum(w * w, axis=1)[None, :]
    whi_ref[...] = w.astype(jnp.bfloat16)


def _vq_kernel(x_ref, w2_ref, whi_ref, enc_ref, q_ref,
               loss_ref, perp_ref, sumsq_ref, counts_ref):
    i = pl.program_id(0)
    nsteps = pl.num_programs(0)
    x = x_ref[...]

    xw = jax.lax.dot_general(
        x.astype(jnp.bfloat16), whi_ref[...],
        dimension_numbers=(((1,), (1,)), ((), ())),
        preferred_element_type=jnp.float32)
    x2 = jnp.sum(x * x, axis=1, keepdims=True)
    dist = x2 + w2_ref[...] - 2.0 * xw
    code_iota = jax.lax.broadcasted_iota(jnp.int32, (BT, NUM_EMB), 1)
    mn = jnp.min(dist, axis=1, keepdims=True)
    masked_iota = jnp.where(dist == mn, code_iota, NUM_EMB)
    idx = jnp.min(masked_iota, axis=1)

    onehot = (code_iota == idx[:, None]).astype(jnp.float32)
    enc_ref[...] = onehot

    ob = onehot.astype(jnp.bfloat16)
    dims = (((1,), (0,)), ((), ()))
    q = jax.lax.dot_general(ob, whi_ref[...], dims,
                            preferred_element_type=jnp.float32)
    q_ref[...] = x + (q - x)

    blk_sumsq = jnp.sum(mn)
    blk_counts = jnp.sum(onehot, axis=0)[None, :]

    @pl.when(i == 0)
    def _init():
        sumsq_ref[0, 0] = blk_sumsq
        counts_ref[...] = blk_counts

    @pl.when(i > 0)
    def _acc():
        sumsq_ref[0, 0] += blk_sumsq
        counts_ref[...] += blk_counts

    @pl.when(i == nsteps - 1)
    def _finalize():
        n_elem = jnp.float32(TOKENS * DIM)
        loss_ref[...] = jnp.full((1, 1), 1.25 * sumsq_ref[0, 0] / n_elem,
                                 dtype=jnp.float32)
        probs = counts_ref[...] / jnp.float32(TOKENS)
        ent = jnp.sum(probs * jnp.log(probs + 1e-10))
        perp_ref[...] = jnp.full((1, 1), jnp.exp(-ent), dtype=jnp.float32)


@functools.partial(jax.jit, static_argnames=())
def _vq(flat_x, weight):
    w2, w_hi = pl.pallas_call(
        _prep_kernel,
        out_shape=[
            jax.ShapeDtypeStruct((1, NUM_EMB), jnp.float32),
            jax.ShapeDtypeStruct((NUM_EMB, DIM), jnp.bfloat16),
        ],
    )(weight)

    grid = (TOKENS // BT,)
    enc, q, loss, perp = pl.pallas_call(
        _vq_kernel,
        grid=grid,
        in_specs=[
            pl.BlockSpec((BT, DIM), lambda i: (i, 0)),
            pl.BlockSpec((1, NUM_EMB), lambda i: (0, 0)),
            pl.BlockSpec((NUM_EMB, DIM), lambda i: (0, 0)),
        ],
        out_specs=[
            pl.BlockSpec((BT, NUM_EMB), lambda i: (i, 0)),
            pl.BlockSpec((BT, DIM), lambda i: (i, 0)),
            pl.BlockSpec((1, 1), lambda i: (0, 0)),
            pl.BlockSpec((1, 1), lambda i: (0, 0)),
        ],
        out_shape=[
            jax.ShapeDtypeStruct((TOKENS, NUM_EMB), jnp.float32),
            jax.ShapeDtypeStruct((TOKENS, DIM), jnp.float32),
            jax.ShapeDtypeStruct((1, 1), jnp.float32),
            jax.ShapeDtypeStruct((1, 1), jnp.float32),
        ],
        scratch_shapes=[
            pltpu.SMEM((1, 1), jnp.float32),
            pltpu.VMEM((1, NUM_EMB), jnp.float32),
        ],
    )(flat_x, w2, w_hi)
    return enc, q, loss, perp


def kernel(inputs, weight):
    flat_x = inputs.reshape(-1, DIM)
    enc, q, loss, perp = _vq(flat_x, weight)
    quantized = q.reshape(inputs.shape)
    return (loss[0, 0], quantized, perp[0, 0], enc)
